# baseline (speedup 1.0000x reference)
import jax
import jax.numpy as jnp
from jax import lax
from jax.experimental import pallas as pl
from jax.experimental.pallas import tpu as pltpu
from jax.experimental.pallas import tpu_sc as plsc

N = 10000
E = 320000
D = 128

NC = 2
NS = 16
NW = NC * NS
EPT = E // NW
EPTP = 10240
C = 64
SCH = 32
NSS = EPTP // (C * SCH)
RING = 2
RPT = 624

_ROWS = 1000


def _dense1_body(x_ref, w_ref, al_ref, ar_ref, h_ref, el_ref, er_ref):
    h = jnp.dot(x_ref[...], w_ref[...], preferred_element_type=jnp.float32)
    h_ref[...] = h
    el_ref[...] = h @ al_ref[...]
    er_ref[...] = h @ ar_ref[...]


def _dense1(x, W, al, ar):
    return pl.pallas_call(
        _dense1_body,
        grid=(N // _ROWS,),
        in_specs=[
            pl.BlockSpec((_ROWS, D), lambda i: (i, 0)),
            pl.BlockSpec((D, D), lambda i: (0, 0)),
            pl.BlockSpec((D, 1), lambda i: (0, 0)),
            pl.BlockSpec((D, 1), lambda i: (0, 0)),
        ],
        out_specs=[
            pl.BlockSpec((_ROWS, D), lambda i: (i, 0)),
            pl.BlockSpec((_ROWS, 1), lambda i: (i, 0)),
            pl.BlockSpec((_ROWS, 1), lambda i: (i, 0)),
        ],
        out_shape=[
            jax.ShapeDtypeStruct((N, D), jnp.float32),
            jax.ShapeDtypeStruct((N, 1), jnp.float32),
            jax.ShapeDtypeStruct((N, 1), jnp.float32),
        ],
    )(x, W, al[:, None], ar[:, None])


def _denred_body(den_ref, out_ref):
    out_ref[...] = jnp.sum(den_ref[...], axis=0)[:, None] + 1e-9


def _denred(den):
    return pl.pallas_call(
        _denred_body,
        grid=(1,),
        in_specs=[pl.BlockSpec((NW, N), lambda i: (0, 0))],
        out_specs=pl.BlockSpec((N, 1), lambda i: (0, 0)),
        out_shape=jax.ShapeDtypeStruct((N, 1), jnp.float32),
    )(den)


def _dense2_body(acc_ref, den_ref, w_ref, al_ref, ar_ref,
                 h_ref, el_ref, er_ref):
    x = (acc_ref[0] + acc_ref[1]) / den_ref[...]
    x = jnp.maximum(x, 0.0)
    h = jnp.dot(x, w_ref[...], preferred_element_type=jnp.float32)
    h_ref[...] = h
    el_ref[...] = h @ al_ref[...]
    er_ref[...] = h @ ar_ref[...]


def _dense2(acc, den, W, al, ar):
    return pl.pallas_call(
        _dense2_body,
        grid=(N // _ROWS,),
        in_specs=[
            pl.BlockSpec((NC, _ROWS, D), lambda i: (0, i, 0)),
            pl.BlockSpec((_ROWS, 1), lambda i: (i, 0)),
            pl.BlockSpec((D, D), lambda i: (0, 0)),
            pl.BlockSpec((D, 1), lambda i: (0, 0)),
            pl.BlockSpec((D, 1), lambda i: (0, 0)),
        ],
        out_specs=[
            pl.BlockSpec((_ROWS, D), lambda i: (i, 0)),
            pl.BlockSpec((_ROWS, 1), lambda i: (i, 0)),
            pl.BlockSpec((_ROWS, 1), lambda i: (i, 0)),
        ],
        out_shape=[
            jax.ShapeDtypeStruct((N, D), jnp.float32),
            jax.ShapeDtypeStruct((N, 1), jnp.float32),
            jax.ShapeDtypeStruct((N, 1), jnp.float32),
        ],
    )(acc, den, W, al[:, None], ar[:, None])


def _combine_body(acc_ref, den_ref, out_ref):
    out_ref[...] = (acc_ref[0] + acc_ref[1]) / den_ref[...]


def _combine(acc, den):
    return pl.pallas_call(
        _combine_body,
        grid=(N // _ROWS,),
        in_specs=[
            pl.BlockSpec((NC, _ROWS, D), lambda i: (0, i, 0)),
            pl.BlockSpec((_ROWS, 1), lambda i: (i, 0)),
        ],
        out_specs=pl.BlockSpec((_ROWS, D), lambda i: (i, 0)),
        out_shape=jax.ShapeDtypeStruct((N, D), jnp.float32),
    )(acc, den)


def _full16(v):
    return jnp.full((16,), v, dtype=jnp.int32)


_GDN = lax.GatherDimensionNumbers(
    offset_dims=(), collapsed_slice_dims=(0,), start_index_map=(0,))


def _lane_bcast(v16, lane):
    idx = jnp.full((16, 1), lane, dtype=jnp.int32)
    return lax.gather(v16, idx, _GDN, (1,),
                      mode=lax.GatherScatterMode.PROMISE_IN_BOUNDS)


def _sc_w_body(el_hbm, er_hbm, src_hbm, dst_hbm, z1_hbm,
               w_out, den_out,
               el_v, er_v, src_v, dst_v, denom_v, w_v):
    cid = lax.axis_index("c")
    sid = lax.axis_index("s")
    wid = sid * NC + cid

    pltpu.sync_copy(el_hbm, el_v)
    pltpu.sync_copy(er_hbm, er_v)
    pltpu.sync_copy(src_hbm.at[wid], src_v)
    pltpu.sync_copy(dst_hbm.at[wid], dst_v)
    pltpu.sync_copy(z1_hbm, denom_v)

    def grp(i, c):
        s16 = src_v[pl.ds(i * 16, 16)]
        d16 = dst_v[pl.ds(i * 16, 16)]
        els = plsc.load_gather(el_v, [s16])
        erd = plsc.load_gather(er_v, [d16])
        x = els + erd
        w16 = jnp.exp(jnp.maximum(x, 0.2 * x))
        w_v[pl.ds(i * 16, 16)] = w16
        plsc.addupdate_scatter(denom_v, [d16], w16)
        return c

    lax.fori_loop(0, EPT // 16, grp, 0)
    z16 = jnp.zeros((16,), jnp.float32)
    for i in range((EPTP - EPT) // 16):
        w_v[pl.ds(EPT + i * 16, 16)] = z16
    pltpu.sync_copy(w_v, w_out.at[wid])
    pltpu.sync_copy(denom_v, den_out.at[wid])


def _sc_w(el, er, src_flat, dst_flat, z1):
    mesh = plsc.VectorSubcoreMesh(core_axis_name="c", subcore_axis_name="s")
    f = pl.kernel(
        _sc_w_body,
        out_type=[
            jax.ShapeDtypeStruct((NW, EPTP), jnp.float32),
            jax.ShapeDtypeStruct((NW, N), jnp.float32),
        ],
        mesh=mesh,
        compiler_params=pltpu.CompilerParams(needs_layout_passes=False),
        scratch_types=[
            pltpu.VMEM((N,), jnp.float32),
            pltpu.VMEM((N,), jnp.float32),
            pltpu.VMEM((EPT,), jnp.int32),
            pltpu.VMEM((EPT,), jnp.int32),
            pltpu.VMEM((N,), jnp.float32),
            pltpu.VMEM((EPTP,), jnp.float32),
        ],
    )
    return f(el, er, src_flat, dst_flat, z1)


def _sc_agg_body(h_hbm, w_hbm, src_hbm, dst_hbm, z2_hbm,
                 acc_out,
                 src_v, dst_v, w_v,
                 g0, g1, s0, s1, d0, d1,
                 gsem, ssem, acc_sh):
    cid = lax.axis_index("c")
    sid = lax.axis_index("s")
    wid = sid * NC + cid
    gbufs = (g0, g1)
    sbufs = (s0, s1)
    dbufs = (d0, d1)

    row0 = pl.multiple_of(sid * RPT, 16)
    last = sid == NS - 1

    @pl.when(last)
    def _():
        pltpu.sync_copy(z2_hbm, acc_sh.at[pl.ds(row0, RPT + 16)])

    @pl.when(jnp.logical_not(last))
    def _():
        pltpu.sync_copy(z2_hbm.at[pl.ds(0, RPT)], acc_sh.at[pl.ds(row0, RPT)])

    plsc.subcore_barrier()

    iota16 = lax.iota(jnp.int32, 16)

    def _rc(ch):
        r = lax.shift_right_logical(ch, 1)
        col = pl.multiple_of(lax.shift_left(jnp.bitwise_and(ch, 1), 6), 64)
        return r, col

    NSPL = 4

    def issue_gather(ch, p):
        r, col = _rc(ch)
        q = C // NSPL
        for s in range(NSPL):
            pltpu.async_copy(
                h_hbm.at[src_v.at[r, pl.ds(col + s * q, q)]],
                gbufs[p].at[pl.ds(s * q, q)], gsem.at[p])

    def wait_gather(p):
        q = C // NSPL
        for s in range(NSPL):
            pltpu.make_async_copy(h_hbm.at[src_v.at[0, pl.ds(0, q)]],
                                  gbufs[p].at[pl.ds(s * q, q)],
                                  gsem.at[p]).wait()

    def issue_scatter(p):
        pltpu.async_copy(sbufs[p], acc_sh.at[dbufs[p].at[0]], ssem.at[p],
                         add=True)

    def wait_scatter(p):
        pltpu.make_async_copy(sbufs[p], acc_sh.at[dbufs[p].at[0]],
                              ssem.at[p]).wait()

    def scale(ch, p):
        gbuf, sbuf = gbufs[p], sbufs[p]
        r_, col = _rc(ch)
        w16s = [w_v[r_, pl.ds(col + g * 16, 16)] for g in range(C // 16)]
        for r in range(C):
            wr = _lane_bcast(w16s[r // 16], r % 16)
            for k in range(D // 16):
                sbuf[r, pl.ds(k * 16, 16)] = gbuf[r, pl.ds(k * 16, 16)] * wr

    def superchunk(ss, c):
        pltpu.sync_copy(src_hbm.at[wid, ss], src_v)
        pltpu.sync_copy(dst_hbm.at[wid, ss], dst_v)
        pltpu.sync_copy(w_hbm.at[wid, ss], w_v)
        for p in range(RING):
            issue_gather(p, p)

        def ring_it(it, c2):
            for p in range(RING):
                ch = it * RING + p
                wait_gather(p)

                @pl.when(it > 0)
                def _():
                    wait_scatter(p)

                r, col = _rc(ch)
                for g in range(C // 16):
                    dbufs[p][0, pl.ds(g * 16, 16)] = (
                        dst_v[r, pl.ds(col + g * 16, 16)])
                scale(ch, p)
                issue_scatter(p)

                @pl.when(ch + RING < SCH)
                def _():
                    issue_gather(ch + RING, p)
            return c2

        lax.fori_loop(0, SCH // RING, ring_it, 0)
        for p in range(RING):
            wait_scatter(p)
        return c

    lax.fori_loop(0, NSS, superchunk, 0)

    plsc.subcore_barrier()

    @pl.when(last)
    def _():
        pltpu.sync_copy(acc_sh.at[pl.ds(row0, RPT + 16)],
                        acc_out.at[cid, pl.ds(row0, RPT + 16)])

    @pl.when(jnp.logical_not(last))
    def _():
        pltpu.sync_copy(acc_sh.at[pl.ds(row0, RPT)],
                        acc_out.at[cid, pl.ds(row0, RPT)])


def _sc_agg(h, w, src_r, dst_r, z2):
    mesh = plsc.VectorSubcoreMesh(core_axis_name="c", subcore_axis_name="s")
    f = pl.kernel(
        _sc_agg_body,
        out_type=[
            jax.ShapeDtypeStruct((NC, N, D), jnp.float32),
        ],
        mesh=mesh,
        compiler_params=pltpu.CompilerParams(needs_layout_passes=False),
        scratch_types=[
            pltpu.VMEM((16, 128), jnp.int32),
            pltpu.VMEM((16, 128), jnp.int32),
            pltpu.VMEM((16, 128), jnp.float32),
        ] + [pltpu.VMEM((C, D), jnp.float32) for _ in range(2 * RING)] + [
            pltpu.VMEM((8, C), jnp.int32) for _ in range(RING)
        ] + [
            pltpu.SemaphoreType.DMA((RING,)),
            pltpu.SemaphoreType.DMA((RING,)),
            pltpu.VMEM_SHARED((N, D), jnp.float32),
        ],
    )
    return f(h, w, src_r, dst_r, z2)


def _sc_edge(h, el, er, src_flat, dst_flat, src_r, dst_r, z1, z2):
    w, den = _sc_w(el, er, src_flat, dst_flat, z1)
    acc = _sc_agg(h, w.reshape(NW, NSS, 16, 128), src_r, dst_r, z2)[0]
    return acc, _denred(den)


def kernel(feature, edge_index, W1, al1, ar1, W2, al2, ar2, W3, al3, ar3):
    src_flat = edge_index[0].reshape(NW, EPT)
    dst_flat = edge_index[1].reshape(NW, EPT)
    pad = jnp.zeros((NW, EPTP - EPT), jnp.int32)
    src_r = jnp.concatenate([src_flat, pad], axis=1).reshape(NW, NSS, 16, 128)
    dst_r = jnp.concatenate([dst_flat, pad], axis=1).reshape(NW, NSS, 16, 128)
    z1 = jnp.zeros((N,), jnp.float32)
    z2 = jnp.zeros((RPT + 16, D), jnp.float32)

    h, el, er = _dense1(feature, W1, al1, ar1)
    acc, den = _sc_edge(h, el.reshape(N), er.reshape(N),
                        src_flat, dst_flat, src_r, dst_r, z1, z2)
    h, el, er = _dense2(acc, den, W2, al2, ar2)
    acc, den = _sc_edge(h, el.reshape(N), er.reshape(N),
                        src_flat, dst_flat, src_r, dst_r, z1, z2)
    h, el, er = _dense2(acc, den, W3, al3, ar3)
    acc, den = _sc_edge(h, el.reshape(N), er.reshape(N),
                        src_flat, dst_flat, src_r, dst_r, z1, z2)
    return _combine(acc, den)


# speedup vs baseline: 1.0268x; 1.0268x over previous
import jax
import jax.numpy as jnp
from jax import lax
from jax.experimental import pallas as pl
from jax.experimental.pallas import tpu as pltpu
from jax.experimental.pallas import tpu_sc as plsc

N = 10000
E = 320000
D = 128

NC = 2
NS = 16
NW = NC * NS
EPT = E // NW
EPTP = 10240
C = 32
SCH = 64
NSS = EPTP // (C * SCH)
GRING = 4
SRING = 2
RPT = 624

_ROWS = 1000


def _dense1_body(x_ref, w_ref, al_ref, ar_ref, h_ref, el_ref, er_ref):
    h = jnp.dot(x_ref[...], w_ref[...], preferred_element_type=jnp.float32)
    h_ref[...] = h
    el_ref[...] = h @ al_ref[...]
    er_ref[...] = h @ ar_ref[...]


def _dense1(x, W, al, ar):
    return pl.pallas_call(
        _dense1_body,
        grid=(N // _ROWS,),
        in_specs=[
            pl.BlockSpec((_ROWS, D), lambda i: (i, 0)),
            pl.BlockSpec((D, D), lambda i: (0, 0)),
            pl.BlockSpec((D, 1), lambda i: (0, 0)),
            pl.BlockSpec((D, 1), lambda i: (0, 0)),
        ],
        out_specs=[
            pl.BlockSpec((_ROWS, D), lambda i: (i, 0)),
            pl.BlockSpec((_ROWS, 1), lambda i: (i, 0)),
            pl.BlockSpec((_ROWS, 1), lambda i: (i, 0)),
        ],
        out_shape=[
            jax.ShapeDtypeStruct((N, D), jnp.float32),
            jax.ShapeDtypeStruct((N, 1), jnp.float32),
            jax.ShapeDtypeStruct((N, 1), jnp.float32),
        ],
    )(x, W, al[:, None], ar[:, None])


def _denred_body(den_ref, out_ref):
    out_ref[...] = jnp.sum(den_ref[...], axis=0)[:, None] + 1e-9


def _denred(den):
    return pl.pallas_call(
        _denred_body,
        grid=(1,),
        in_specs=[pl.BlockSpec((NW, N), lambda i: (0, 0))],
        out_specs=pl.BlockSpec((N, 1), lambda i: (0, 0)),
        out_shape=jax.ShapeDtypeStruct((N, 1), jnp.float32),
    )(den)


def _dense2_body(acc_ref, den_ref, w_ref, al_ref, ar_ref,
                 h_ref, el_ref, er_ref):
    x = (acc_ref[0] + acc_ref[1]) / den_ref[...]
    x = jnp.maximum(x, 0.0)
    h = jnp.dot(x, w_ref[...], preferred_element_type=jnp.float32)
    h_ref[...] = h
    el_ref[...] = h @ al_ref[...]
    er_ref[...] = h @ ar_ref[...]


def _dense2(acc, den, W, al, ar):
    return pl.pallas_call(
        _dense2_body,
        grid=(N // _ROWS,),
        in_specs=[
            pl.BlockSpec((NC, _ROWS, D), lambda i: (0, i, 0)),
            pl.BlockSpec((_ROWS, 1), lambda i: (i, 0)),
            pl.BlockSpec((D, D), lambda i: (0, 0)),
            pl.BlockSpec((D, 1), lambda i: (0, 0)),
            pl.BlockSpec((D, 1), lambda i: (0, 0)),
        ],
        out_specs=[
            pl.BlockSpec((_ROWS, D), lambda i: (i, 0)),
            pl.BlockSpec((_ROWS, 1), lambda i: (i, 0)),
            pl.BlockSpec((_ROWS, 1), lambda i: (i, 0)),
        ],
        out_shape=[
            jax.ShapeDtypeStruct((N, D), jnp.float32),
            jax.ShapeDtypeStruct((N, 1), jnp.float32),
            jax.ShapeDtypeStruct((N, 1), jnp.float32),
        ],
    )(acc, den, W, al[:, None], ar[:, None])


def _combine_body(acc_ref, den_ref, out_ref):
    out_ref[...] = (acc_ref[0] + acc_ref[1]) / den_ref[...]


def _combine(acc, den):
    return pl.pallas_call(
        _combine_body,
        grid=(N // _ROWS,),
        in_specs=[
            pl.BlockSpec((NC, _ROWS, D), lambda i: (0, i, 0)),
            pl.BlockSpec((_ROWS, 1), lambda i: (i, 0)),
        ],
        out_specs=pl.BlockSpec((_ROWS, D), lambda i: (i, 0)),
        out_shape=jax.ShapeDtypeStruct((N, D), jnp.float32),
    )(acc, den)


def _full16(v):
    return jnp.full((16,), v, dtype=jnp.int32)


_GDN = lax.GatherDimensionNumbers(
    offset_dims=(), collapsed_slice_dims=(0,), start_index_map=(0,))


def _lane_bcast(v16, lane):
    idx = jnp.full((16, 1), lane, dtype=jnp.int32)
    return lax.gather(v16, idx, _GDN, (1,),
                      mode=lax.GatherScatterMode.PROMISE_IN_BOUNDS)


def _sc_w_body(el_hbm, er_hbm, src_hbm, dst_hbm, z1_hbm,
               w_out, den_out,
               el_v, er_v, src_v, dst_v, denom_v, w_v):
    cid = lax.axis_index("c")
    sid = lax.axis_index("s")
    wid = sid * NC + cid

    pltpu.sync_copy(el_hbm, el_v)
    pltpu.sync_copy(er_hbm, er_v)
    pltpu.sync_copy(src_hbm.at[wid], src_v)
    pltpu.sync_copy(dst_hbm.at[wid], dst_v)
    pltpu.sync_copy(z1_hbm, denom_v)

    def grp(i, c):
        s16 = src_v[pl.ds(i * 16, 16)]
        d16 = dst_v[pl.ds(i * 16, 16)]
        els = plsc.load_gather(el_v, [s16])
        erd = plsc.load_gather(er_v, [d16])
        x = els + erd
        w16 = jnp.exp(jnp.maximum(x, 0.2 * x))
        w_v[pl.ds(i * 16, 16)] = w16
        plsc.addupdate_scatter(denom_v, [d16], w16)
        return c

    lax.fori_loop(0, EPT // 16, grp, 0)
    z16 = jnp.zeros((16,), jnp.float32)
    for i in range((EPTP - EPT) // 16):
        w_v[pl.ds(EPT + i * 16, 16)] = z16
    pltpu.sync_copy(w_v, w_out.at[wid])
    pltpu.sync_copy(denom_v, den_out.at[wid])


def _sc_w(el, er, src_flat, dst_flat, z1):
    mesh = plsc.VectorSubcoreMesh(core_axis_name="c", subcore_axis_name="s")
    f = pl.kernel(
        _sc_w_body,
        out_type=[
            jax.ShapeDtypeStruct((NW, EPTP), jnp.float32),
            jax.ShapeDtypeStruct((NW, N), jnp.float32),
        ],
        mesh=mesh,
        compiler_params=pltpu.CompilerParams(needs_layout_passes=False),
        scratch_types=[
            pltpu.VMEM((N,), jnp.float32),
            pltpu.VMEM((N,), jnp.float32),
            pltpu.VMEM((EPT,), jnp.int32),
            pltpu.VMEM((EPT,), jnp.int32),
            pltpu.VMEM((N,), jnp.float32),
            pltpu.VMEM((EPTP,), jnp.float32),
        ],
    )
    return f(el, er, src_flat, dst_flat, z1)


def _sc_agg_body(h_hbm, w_hbm, src_hbm, dst_hbm, z2_hbm,
                 acc_out,
                 src_v, dst_v, w_v,
                 g0, g1, g2, g3, s0, s1, d0, d1,
                 gsem, ssem, acc_sh):
    cid = lax.axis_index("c")
    sid = lax.axis_index("s")
    wid = sid * NC + cid
    gbufs = (g0, g1, g2, g3)
    sbufs = (s0, s1)
    dbufs = (d0, d1)

    row0 = pl.multiple_of(sid * RPT, 16)
    last = sid == NS - 1

    @pl.when(last)
    def _():
        pltpu.sync_copy(z2_hbm, acc_sh.at[pl.ds(row0, RPT + 16)])

    @pl.when(jnp.logical_not(last))
    def _():
        pltpu.sync_copy(z2_hbm.at[pl.ds(0, RPT)], acc_sh.at[pl.ds(row0, RPT)])

    plsc.subcore_barrier()

    iota16 = lax.iota(jnp.int32, 16)

    def _rc(ch):
        r = lax.shift_right_logical(ch, 2)
        col = pl.multiple_of(lax.shift_left(jnp.bitwise_and(ch, 3), 5), 32)
        return r, col

    NSPL = 2

    def issue_gather(ch, p):
        r, col = _rc(ch)
        q = C // NSPL
        for s in range(NSPL):
            pltpu.async_copy(
                h_hbm.at[src_v.at[r, pl.ds(col + s * q, q)]],
                gbufs[p].at[pl.ds(s * q, q)], gsem.at[p])

    def wait_gather(p):
        q = C // NSPL
        for s in range(NSPL):
            pltpu.make_async_copy(h_hbm.at[src_v.at[0, pl.ds(0, q)]],
                                  gbufs[p].at[pl.ds(s * q, q)],
                                  gsem.at[p]).wait()

    def issue_scatter(p):
        pltpu.async_copy(sbufs[p], acc_sh.at[dbufs[p].at[0]], ssem.at[p],
                         add=True)

    def wait_scatter(p):
        pltpu.make_async_copy(sbufs[p], acc_sh.at[dbufs[p].at[0]],
                              ssem.at[p]).wait()

    def scale(ch, gp, sp):
        gbuf, sbuf = gbufs[gp], sbufs[sp]
        r_, col = _rc(ch)
        w16s = [w_v[r_, pl.ds(col + g * 16, 16)] for g in range(C // 16)]
        for r in range(C):
            wr = _lane_bcast(w16s[r // 16], r % 16)
            for k in range(D // 16):
                sbuf[r, pl.ds(k * 16, 16)] = gbuf[r, pl.ds(k * 16, 16)] * wr

    def superchunk(ss, c):
        pltpu.sync_copy(src_hbm.at[wid, ss], src_v)
        pltpu.sync_copy(dst_hbm.at[wid, ss], dst_v)
        pltpu.sync_copy(w_hbm.at[wid, ss], w_v)
        for p in range(GRING):
            issue_gather(p, p)

        def ring_it(it, c2):
            for j in range(GRING):
                ch = it * GRING + j
                gp = j
                sp = j % SRING
                wait_gather(gp)

                @pl.when(ch >= SRING)
                def _():
                    wait_scatter(sp)

                r, col = _rc(ch)
                for g in range(C // 16):
                    dbufs[sp][0, pl.ds(g * 16, 16)] = (
                        dst_v[r, pl.ds(col + g * 16, 16)])
                scale(ch, gp, sp)
                issue_scatter(sp)

                @pl.when(ch + GRING < SCH)
                def _():
                    issue_gather(ch + GRING, gp)
            return c2

        lax.fori_loop(0, SCH // GRING, ring_it, 0)
        for p in range(SRING):
            wait_scatter(p)
        return c

    lax.fori_loop(0, NSS, superchunk, 0)

    plsc.subcore_barrier()

    @pl.when(last)
    def _():
        pltpu.sync_copy(acc_sh.at[pl.ds(row0, RPT + 16)],
                        acc_out.at[cid, pl.ds(row0, RPT + 16)])

    @pl.when(jnp.logical_not(last))
    def _():
        pltpu.sync_copy(acc_sh.at[pl.ds(row0, RPT)],
                        acc_out.at[cid, pl.ds(row0, RPT)])


def _sc_agg(h, w, src_r, dst_r, z2):
    mesh = plsc.VectorSubcoreMesh(core_axis_name="c", subcore_axis_name="s")
    f = pl.kernel(
        _sc_agg_body,
        out_type=[
            jax.ShapeDtypeStruct((NC, N, D), jnp.float32),
        ],
        mesh=mesh,
        compiler_params=pltpu.CompilerParams(needs_layout_passes=False),
        scratch_types=[
            pltpu.VMEM((16, 128), jnp.int32),
            pltpu.VMEM((16, 128), jnp.int32),
            pltpu.VMEM((16, 128), jnp.float32),
        ] + [pltpu.VMEM((C, D), jnp.float32) for _ in range(GRING + SRING)] + [
            pltpu.VMEM((8, C), jnp.int32) for _ in range(SRING)
        ] + [
            pltpu.SemaphoreType.DMA((GRING,)),
            pltpu.SemaphoreType.DMA((SRING,)),
            pltpu.VMEM_SHARED((N, D), jnp.float32),
        ],
    )
    return f(h, w, src_r, dst_r, z2)


def _sc_edge(h, el, er, src_flat, dst_flat, src_r, dst_r, z1, z2):
    w, den = _sc_w(el, er, src_flat, dst_flat, z1)
    acc = _sc_agg(h, w.reshape(NW, NSS, 16, 128), src_r, dst_r, z2)[0]
    return acc, _denred(den)


def kernel(feature, edge_index, W1, al1, ar1, W2, al2, ar2, W3, al3, ar3):
    src_flat = edge_index[0].reshape(NW, EPT)
    dst_flat = edge_index[1].reshape(NW, EPT)
    pad = jnp.zeros((NW, EPTP - EPT), jnp.int32)
    src_r = jnp.concatenate([src_flat, pad], axis=1).reshape(NW, NSS, 16, 128)
    dst_r = jnp.concatenate([dst_flat, pad], axis=1).reshape(NW, NSS, 16, 128)
    z1 = jnp.zeros((N,), jnp.float32)
    z2 = jnp.zeros((RPT + 16, D), jnp.float32)

    h, el, er = _dense1(feature, W1, al1, ar1)
    acc, den = _sc_edge(h, el.reshape(N), er.reshape(N),
                        src_flat, dst_flat, src_r, dst_r, z1, z2)
    h, el, er = _dense2(acc, den, W2, al2, ar2)
    acc, den = _sc_edge(h, el.reshape(N), er.reshape(N),
                        src_flat, dst_flat, src_r, dst_r, z1, z2)
    h, el, er = _dense2(acc, den, W3, al3, ar3)
    acc, den = _sc_edge(h, el.reshape(N), er.reshape(N),
                        src_flat, dst_flat, src_r, dst_r, z1, z2)
    return _combine(acc, den)
